# baseline (speedup 1.0000x reference)
import functools

import jax
import jax.numpy as jnp
from jax import lax
from jax.experimental import pallas as pl
from jax.experimental.pallas import tpu as pltpu
from jax.experimental.pallas import tpu_sc as plsc

N = 41472
E = 829440
H = 96
EMB = 16
DE = 16

_GW = 32
_GCHUNK = (2 * E) // _GW
_GSUB = 6480
_GNSUB = _GCHUNK // _GSUB


def _gather_cls(nodes, eflat):
    mesh = plsc.VectorSubcoreMesh(core_axis_name="c", subcore_axis_name="s")

    @functools.partial(
        pl.kernel,
        out_type=jax.ShapeDtypeStruct((2 * E,), jnp.int32),
        mesh=mesh,
        scratch_types=[
            pltpu.VMEM_SHARED((N,), jnp.int32),
            pltpu.VMEM((_GSUB,), jnp.int32),
            pltpu.VMEM((_GSUB,), jnp.int32),
            pltpu.VMEM((_GSUB,), jnp.int32),
            pltpu.VMEM((_GSUB,), jnp.int32),
            pltpu.SemaphoreType.DMA((2,)),
            pltpu.SemaphoreType.DMA,
        ],
    )
    def k(nodes_hbm, eflat_hbm, out_hbm, tbl, ibuf0, ibuf1, obuf0, obuf1,
          isem, gsem):
        ibufs = (ibuf0, ibuf1)
        obufs = (obuf0, obuf1)
        cid = lax.axis_index("c")
        sid = lax.axis_index("s")
        wid = sid * 2 + cid
        base = wid * _GCHUNK

        @pl.when(sid == 0)
        def _():
            pltpu.sync_copy(nodes_hbm, tbl)
        plsc.subcore_barrier()

        pltpu.async_copy(eflat_hbm.at[pl.ds(base, _GSUB)], ibufs[0],
                         isem.at[0])
        for sc in range(_GNSUB):
            b = sc % 2
            off = base + sc * _GSUB
            if sc + 1 < _GNSUB:
                pltpu.async_copy(
                    eflat_hbm.at[pl.ds(off + _GSUB, _GSUB)],
                    ibufs[1 - b], isem.at[1 - b])
            pltpu.make_async_copy(eflat_hbm.at[pl.ds(off, _GSUB)],
                                  ibufs[b], isem.at[b]).wait()
            pltpu.async_copy(tbl.at[ibufs[b]], obufs[b], gsem).wait()
            pltpu.sync_copy(obufs[b], out_hbm.at[pl.ds(off, _GSUB)])

    return k(nodes, eflat)


_SG = 24
_SCHUNK_ROWS = 9
_SCHUNK = _SCHUNK_ROWS * 128
_SUPER_ROWS = 16 * _SCHUNK_ROWS
_NSUPER = (E // 128) // _SUPER_ROWS
_CHUNKS = (12, 12, 12, 9)
_SZROWS = 162
_NPT = N // 16


def _scatter_agg(dst2d, msgs2, super0, nsup):
    mesh = plsc.VectorSubcoreMesh(core_axis_name="c", subcore_axis_name="s")

    @functools.partial(
        pl.kernel,
        out_type=jax.ShapeDtypeStruct((N, 128), jnp.float32),
        mesh=mesh,
        scratch_types=[
            pltpu.VMEM_SHARED((N, _SG), jnp.float32),
            pltpu.VMEM((_SZROWS, _SG), jnp.float32),
            pltpu.VMEM((2, _SCHUNK, _SG), jnp.float32),
            pltpu.VMEM((2, _SCHUNK_ROWS, 128), jnp.int32),
            pltpu.SemaphoreType.DMA((2,)),
            pltpu.SemaphoreType.DMA((2,)),
            pltpu.SemaphoreType.DMA,
        ],
        compiler_params=pltpu.CompilerParams(use_tc_tiling_on_sc=False),
    )
    def k(dst2d_hbm, msgs2_hbm, out_hbm, acc, zbuf, dbuf, ibuf, dsem, isem,
          ssem):
        cid = lax.axis_index("c")
        sid = lax.axis_index("s")

        zeros16 = jnp.zeros((16,), jnp.float32)

        def zrow(i, carry):
            zbuf[i, pl.ds(0, 16)] = zeros16
            zbuf[i, pl.ds(8, 16)] = zeros16
            return carry

        lax.fori_loop(0, _SZROWS, zrow, 0)

        for p in range(2):
            grp = cid * 2 + p
            col0 = grp * _SG
            for t in range(_NPT // _SZROWS):
                pltpu.sync_copy(
                    zbuf, acc.at[pl.ds(sid * _NPT + t * _SZROWS, _SZROWS)])
            plsc.subcore_barrier()

            def start_in(t, b):
                lrow0 = t * _SUPER_ROWS + sid * _SCHUNK_ROWS
                grow0 = super0 * _SUPER_ROWS + lrow0
                pltpu.async_copy(
                    msgs2_hbm.at[pl.ds(lrow0 * 128, _SCHUNK),
                                 pl.ds(col0, _SG)],
                    dbuf.at[b], dsem.at[b])
                pltpu.async_copy(dst2d_hbm.at[pl.ds(grow0, _SCHUNK_ROWS)],
                                 ibuf.at[b], isem.at[b])

            def wait_in(t, b):
                lrow0 = t * _SUPER_ROWS + sid * _SCHUNK_ROWS
                grow0 = super0 * _SUPER_ROWS + lrow0
                pltpu.make_async_copy(
                    msgs2_hbm.at[pl.ds(lrow0 * 128, _SCHUNK),
                                 pl.ds(col0, _SG)],
                    dbuf.at[b], dsem.at[b]).wait()
                pltpu.make_async_copy(
                    dst2d_hbm.at[pl.ds(grow0, _SCHUNK_ROWS)],
                    ibuf.at[b], isem.at[b]).wait()

            start_in(0, 0)

            def chunk2(t2, carry):
                for b in range(2):
                    t = t2 * 2 + b

                    @pl.when(t < nsup)
                    def _():
                        @pl.when(t + 1 < nsup)
                        def _():
                            start_in(t + 1, 1 - b)
                        wait_in(t, b)
                        descs = []
                        for j in range(_SCHUNK_ROWS):
                            descs.append(pltpu.async_copy(
                                dbuf.at[b, pl.ds(j * 128, 128)],
                                acc.at[ibuf.at[b, j]], ssem, add=True))
                        for d in descs:
                            d.wait()
                return carry

            lax.fori_loop(0, (nsup + 1) // 2, chunk2, 0)
            plsc.subcore_barrier()

            pltpu.sync_copy(acc.at[pl.ds(sid * _NPT, _NPT)],
                            out_hbm.at[pl.ds(sid * _NPT, _NPT),
                                       pl.ds(col0, _SG)])

    return k(dst2d, msgs2)


_EB = 4608
_ENB = E // _EB


def _edge_body(cs_ref, cd_ref, eft_ref, emb_ref, mW0_ref, b0_ref, mW1_ref,
               b1_ref, mW2p_ref, b2p_ref, out_ref):
    f32 = jnp.float32
    bf16 = jnp.bfloat16
    dims11 = (((1,), (1,)), ((), ()))
    dims00 = (((0,), (0,)), ((), ()))
    dims01 = (((0,), (1,)), ((), ()))
    cs = cs_ref[0]
    cd = cd_ref[0]
    iota10c = lax.broadcasted_iota(jnp.int32, (10, 1), 0)
    oh_st = (cs == iota10c).astype(bf16)
    oh_dt = (cd == iota10c).astype(bf16)
    ohcat = jnp.concatenate([oh_st, oh_dt], axis=0)
    emb = emb_ref[...]
    TA = lax.dot_general(emb, mW0_ref[:, 0:EMB], dims11,
                         preferred_element_type=f32)
    TB = lax.dot_general(emb, mW0_ref[:, EMB:2 * EMB], dims11,
                         preferred_element_type=f32)
    TAB = jnp.concatenate([TA, TB], axis=0)
    h0 = (lax.dot_general(ohcat, TAB.astype(bf16), dims00,
                          preferred_element_type=f32)
          + lax.dot_general(eft_ref[...].astype(bf16),
                            mW0_ref[:, 2 * EMB:].astype(bf16), dims01,
                            preferred_element_type=f32)
          + b0_ref[...])
    h1 = jnp.maximum(h0, 0.0).astype(bf16)
    h2 = jnp.maximum(
        lax.dot_general(h1, mW1_ref[...].astype(bf16), dims11,
                        preferred_element_type=f32)
        + b1_ref[...], 0.0).astype(bf16)
    out_ref[...] = lax.dot_general(h2, mW2p_ref[...].astype(bf16), dims11,
                                   preferred_element_type=f32) + b2p_ref[...]


def _edge_mlp(cs3, cd3, eft, emb_s, mW0, b0r, mW1, b1r, mW2p, b2pr):
    nb = cs3.shape[0]
    ne = nb * _EB
    full = lambda shape: pl.BlockSpec(shape, lambda i, _s=shape: tuple(0 for _ in _s))
    return pl.pallas_call(
        _edge_body,
        grid=(nb,),
        in_specs=[
            pl.BlockSpec((1, 1, _EB), lambda i: (i, 0, 0)),
            pl.BlockSpec((1, 1, _EB), lambda i: (i, 0, 0)),
            pl.BlockSpec((DE, _EB), lambda i: (0, i)),
            full((10, EMB)),
            full((H, 2 * EMB + DE)),
            full((1, H)),
            full((H, H)),
            full((1, H)),
            full((128, H)),
            full((1, 128)),
        ],
        out_specs=pl.BlockSpec((_EB, 128), lambda i: (i, 0)),
        out_shape=jax.ShapeDtypeStruct((ne, 128), jnp.float32),
        compiler_params=pltpu.CompilerParams(
            dimension_semantics=("arbitrary",)),
    )(cs3, cd3, eft, emb_s, mW0, b0r, mW1, b1r, mW2p, b2pr)


_RB = 2304
_RNB = N // _RB


def _post_body(agg0_ref, agg1_ref, agg2_ref, agg3_ref, pzt_ref, sht_ref,
               sct_ref, pW0_ref, pb0_ref, pW1_ref,
               pb1_ref, pW2_ref, pb2_ref, Wi_ref, Wf_ref, Wg_ref, Wo_ref,
               Ui_ref, Uf_ref, Ug_ref, Uo_ref, bi_ref, bf_ref, bg_ref,
               bo_ref, oW_ref, ob_ref, h_ref, c_ref, o_ref):
    f32 = jnp.float32
    dims11 = (((1,), (1,)), ((), ()))
    dims10 = (((1,), (0,)), ((), ()))

    bf16 = jnp.bfloat16

    def dott(w, x):
        return lax.dot_general(w.astype(bf16), x.astype(bf16), dims10,
                               preferred_element_type=f32)

    agg96 = ((agg0_ref[:, 0:H] + agg1_ref[:, 0:H])
             + (agg2_ref[:, 0:H] + agg3_ref[:, 0:H]))
    g0 = (lax.dot_general(pW0_ref[:, 0:H].astype(bf16), agg96.astype(bf16),
                          dims11, preferred_element_type=f32)
          + dott(pW0_ref[:, H:H + EMB], pzt_ref[...]) + pb0_ref[...])
    h = jnp.maximum(g0, 0.0)
    h = jnp.maximum(dott(pW1_ref[...], h) + pb1_ref[...], 0.0)
    hp = dott(pW2_ref[...], h) + pb2_ref[...]
    sh = sht_ref[...]
    ii = jax.nn.sigmoid(dott(Wi_ref[...], hp) + dott(Ui_ref[...], sh) + bi_ref[...])
    ff = jax.nn.sigmoid(dott(Wf_ref[...], hp) + dott(Uf_ref[...], sh) + bf_ref[...])
    gg = jnp.tanh(dott(Wg_ref[...], hp) + dott(Ug_ref[...], sh) + bg_ref[...])
    oo = jax.nn.sigmoid(dott(Wo_ref[...], hp) + dott(Uo_ref[...], sh) + bo_ref[...])
    cn = ff * sct_ref[...] + ii * gg
    hn = oo * jnp.tanh(cn)
    h_ref[...] = hn
    c_ref[...] = cn
    o_ref[...] = dott(oW_ref[...], hn) + ob_ref[...]


def _post(aggs, puzzlet, sht, sct, pW0, pb0c, pW1, pb1c, pW2, pb2c, Ws, Us,
          bs, oW, obc):
    full = lambda shape: pl.BlockSpec(shape, lambda i, _s=shape: tuple(0 for _ in _s))
    return pl.pallas_call(
        _post_body,
        grid=(_RNB,),
        in_specs=[
            pl.BlockSpec((_RB, 128), lambda i: (i, 0)),
            pl.BlockSpec((_RB, 128), lambda i: (i, 0)),
            pl.BlockSpec((_RB, 128), lambda i: (i, 0)),
            pl.BlockSpec((_RB, 128), lambda i: (i, 0)),
            pl.BlockSpec((EMB, _RB), lambda i: (0, i)),
            pl.BlockSpec((H, _RB), lambda i: (0, i)),
            pl.BlockSpec((H, _RB), lambda i: (0, i)),
            full((H, H + EMB)),
            full((H, 1)),
            full((H, H)),
            full((H, 1)),
            full((H, H)),
            full((H, 1)),
            *[full((H, H)) for _ in range(8)],
            *[full((H, 1)) for _ in range(4)],
            full((10, H)),
            full((10, 1)),
        ],
        out_specs=[
            pl.BlockSpec((H, _RB), lambda i: (0, i)),
            pl.BlockSpec((H, _RB), lambda i: (0, i)),
            pl.BlockSpec((10, _RB), lambda i: (0, i)),
        ],
        out_shape=[
            jax.ShapeDtypeStruct((H, N), jnp.float32),
            jax.ShapeDtypeStruct((H, N), jnp.float32),
            jax.ShapeDtypeStruct((10, N), jnp.float32),
        ],
        compiler_params=pltpu.CompilerParams(
            dimension_semantics=("arbitrary",)),
    )(*aggs, puzzlet, sht, sct, pW0, pb0c, pW1, pb1c, pW2, pb2c, *Ws, *Us,
      *bs, oW, obc)


def kernel(puzzle, nodes, edges, edge_features, state_h, state_c, first, emb,
           mW0, mb0, mW1, mb1, mW2, mb2, pW0, pb0, pW1, pb1, pW2, pb2, W_ih,
           W_hh, b_ih, b_hh, oW, ob):
    f32 = jnp.float32
    nodes = nodes.astype(jnp.int32)
    eflat = edges.astype(jnp.int32).reshape(2 * E)
    dst2d = edges[1].astype(jnp.int32).reshape(E // 128, 128)

    cls = _gather_cls(nodes, eflat)
    cs3 = cls[:E].reshape(_ENB, 1, _EB)
    cd3 = cls[E:].reshape(_ENB, 1, _EB)
    eft = edge_features.T

    emb_s = emb * jnp.asarray(first, f32)
    mW2p = jnp.concatenate([mW2, jnp.zeros((128 - H, H), f32)], axis=0)
    b2p = jnp.concatenate([mb2, jnp.zeros((128 - H,), f32)]).reshape(1, 128)

    aggs = []
    sup0 = 0
    for nsup in _CHUNKS:
        e0 = sup0 * _SUPER_ROWS * 128
        e1 = (sup0 + nsup) * _SUPER_ROWS * 128
        b0_, b1_ = e0 // _EB, e1 // _EB
        msgs_k = _edge_mlp(
            cs3[b0_:b1_], cd3[b0_:b1_],
            eft[:, e0:e1], emb_s, mW0, mb0.reshape(1, H), mW1,
            mb1.reshape(1, H), mW2p, b2p)
        aggs.append(_scatter_agg(dst2d, msgs_k, sup0, nsup))
        sup0 += nsup

    b = b_ih + b_hh
    Ws = [W_ih[i * H:(i + 1) * H] for i in range(4)]
    Us = [W_hh[i * H:(i + 1) * H] for i in range(4)]
    bs = [b[i * H:(i + 1) * H].reshape(H, 1) for i in range(4)]
    ht, ct, outt = _post(
        aggs, puzzle.T, state_h.T, state_c.T, pW0, pb0.reshape(H, 1), pW1,
        pb1.reshape(H, 1), pW2, pb2.reshape(H, 1), Ws, Us, bs, oW,
        ob.reshape(10, 1))
    return (ht.T, ct.T, outt.T.reshape(-1, 81, 10))


# speedup vs baseline: 1.0101x; 1.0101x over previous
import functools

import jax
import jax.numpy as jnp
from jax import lax
from jax.experimental import pallas as pl
from jax.experimental.pallas import tpu as pltpu
from jax.experimental.pallas import tpu_sc as plsc

N = 41472
E = 829440
H = 96
EMB = 16
DE = 16

_GW = 32
_GCHUNK = (2 * E) // _GW
_GSUB = 6480
_GNSUB = _GCHUNK // _GSUB


def _gather_cls(nodes, eflat):
    mesh = plsc.VectorSubcoreMesh(core_axis_name="c", subcore_axis_name="s")

    @functools.partial(
        pl.kernel,
        out_type=jax.ShapeDtypeStruct((2 * E,), jnp.int32),
        mesh=mesh,
        scratch_types=[
            pltpu.VMEM_SHARED((N,), jnp.int32),
            pltpu.VMEM((_GSUB,), jnp.int32),
            pltpu.VMEM((_GSUB,), jnp.int32),
            pltpu.VMEM((_GSUB,), jnp.int32),
            pltpu.VMEM((_GSUB,), jnp.int32),
            pltpu.SemaphoreType.DMA((2,)),
            pltpu.SemaphoreType.DMA,
        ],
    )
    def k(nodes_hbm, eflat_hbm, out_hbm, tbl, ibuf0, ibuf1, obuf0, obuf1,
          isem, gsem):
        ibufs = (ibuf0, ibuf1)
        obufs = (obuf0, obuf1)
        cid = lax.axis_index("c")
        sid = lax.axis_index("s")
        wid = sid * 2 + cid
        base = wid * _GCHUNK

        @pl.when(sid == 0)
        def _():
            pltpu.sync_copy(nodes_hbm, tbl)
        plsc.subcore_barrier()

        pltpu.async_copy(eflat_hbm.at[pl.ds(base, _GSUB)], ibufs[0],
                         isem.at[0])
        for sc in range(_GNSUB):
            b = sc % 2
            off = base + sc * _GSUB
            if sc + 1 < _GNSUB:
                pltpu.async_copy(
                    eflat_hbm.at[pl.ds(off + _GSUB, _GSUB)],
                    ibufs[1 - b], isem.at[1 - b])
            pltpu.make_async_copy(eflat_hbm.at[pl.ds(off, _GSUB)],
                                  ibufs[b], isem.at[b]).wait()
            pltpu.async_copy(tbl.at[ibufs[b]], obufs[b], gsem).wait()
            pltpu.sync_copy(obufs[b], out_hbm.at[pl.ds(off, _GSUB)])

    return k(nodes, eflat)


_SG = 24
_SCHUNK_ROWS = 9
_SCHUNK = _SCHUNK_ROWS * 128
_SUPER_ROWS = 16 * _SCHUNK_ROWS
_NSUPER = (E // 128) // _SUPER_ROWS
_CHUNKS = (15, 15, 15)
_SZROWS = 162
_NPT = N // 16


def _scatter_agg(dst2d, msgs2, super0, nsup):
    mesh = plsc.VectorSubcoreMesh(core_axis_name="c", subcore_axis_name="s")

    @functools.partial(
        pl.kernel,
        out_type=jax.ShapeDtypeStruct((N, 128), jnp.float32),
        mesh=mesh,
        scratch_types=[
            pltpu.VMEM_SHARED((N, _SG), jnp.float32),
            pltpu.VMEM((_SZROWS, _SG), jnp.float32),
            pltpu.VMEM((2, _SCHUNK, _SG), jnp.float32),
            pltpu.VMEM((2, _SCHUNK_ROWS, 128), jnp.int32),
            pltpu.SemaphoreType.DMA((2,)),
            pltpu.SemaphoreType.DMA((2,)),
            pltpu.SemaphoreType.DMA,
        ],
        compiler_params=pltpu.CompilerParams(use_tc_tiling_on_sc=False),
    )
    def k(dst2d_hbm, msgs2_hbm, out_hbm, acc, zbuf, dbuf, ibuf, dsem, isem,
          ssem):
        cid = lax.axis_index("c")
        sid = lax.axis_index("s")

        zeros16 = jnp.zeros((16,), jnp.float32)

        def zrow(i, carry):
            zbuf[i, pl.ds(0, 16)] = zeros16
            zbuf[i, pl.ds(8, 16)] = zeros16
            return carry

        lax.fori_loop(0, _SZROWS, zrow, 0)

        for p in range(2):
            grp = cid * 2 + p
            col0 = grp * _SG
            for t in range(_NPT // _SZROWS):
                pltpu.sync_copy(
                    zbuf, acc.at[pl.ds(sid * _NPT + t * _SZROWS, _SZROWS)])
            plsc.subcore_barrier()

            def start_in(t, b):
                lrow0 = t * _SUPER_ROWS + sid * _SCHUNK_ROWS
                grow0 = super0 * _SUPER_ROWS + lrow0
                pltpu.async_copy(
                    msgs2_hbm.at[pl.ds(lrow0 * 128, _SCHUNK),
                                 pl.ds(col0, _SG)],
                    dbuf.at[b], dsem.at[b])
                pltpu.async_copy(dst2d_hbm.at[pl.ds(grow0, _SCHUNK_ROWS)],
                                 ibuf.at[b], isem.at[b])

            def wait_in(t, b):
                lrow0 = t * _SUPER_ROWS + sid * _SCHUNK_ROWS
                grow0 = super0 * _SUPER_ROWS + lrow0
                pltpu.make_async_copy(
                    msgs2_hbm.at[pl.ds(lrow0 * 128, _SCHUNK),
                                 pl.ds(col0, _SG)],
                    dbuf.at[b], dsem.at[b]).wait()
                pltpu.make_async_copy(
                    dst2d_hbm.at[pl.ds(grow0, _SCHUNK_ROWS)],
                    ibuf.at[b], isem.at[b]).wait()

            start_in(0, 0)

            def chunk2(t2, carry):
                for b in range(2):
                    t = t2 * 2 + b

                    @pl.when(t < nsup)
                    def _():
                        @pl.when(t + 1 < nsup)
                        def _():
                            start_in(t + 1, 1 - b)
                        wait_in(t, b)
                        descs = []
                        for j in range(_SCHUNK_ROWS):
                            descs.append(pltpu.async_copy(
                                dbuf.at[b, pl.ds(j * 128, 128)],
                                acc.at[ibuf.at[b, j]], ssem, add=True))
                        for d in descs:
                            d.wait()
                return carry

            lax.fori_loop(0, (nsup + 1) // 2, chunk2, 0)
            plsc.subcore_barrier()

            pltpu.sync_copy(acc.at[pl.ds(sid * _NPT, _NPT)],
                            out_hbm.at[pl.ds(sid * _NPT, _NPT),
                                       pl.ds(col0, _SG)])

    return k(dst2d, msgs2)


_EB = 5120
_ENB = E // _EB


def _edge_body(cs_ref, cd_ref, eft_ref, emb_ref, mW0_ref, b0_ref, mW1_ref,
               b1_ref, mW2p_ref, b2p_ref, out_ref):
    f32 = jnp.float32
    bf16 = jnp.bfloat16
    dims11 = (((1,), (1,)), ((), ()))
    dims00 = (((0,), (0,)), ((), ()))
    dims01 = (((0,), (1,)), ((), ()))
    cs = cs_ref[0]
    cd = cd_ref[0]
    iota10c = lax.broadcasted_iota(jnp.int32, (10, 1), 0)
    oh_st = (cs == iota10c).astype(bf16)
    oh_dt = (cd == iota10c).astype(bf16)
    ohcat = jnp.concatenate([oh_st, oh_dt], axis=0)
    emb = emb_ref[...]
    TA = lax.dot_general(emb, mW0_ref[:, 0:EMB], dims11,
                         preferred_element_type=f32)
    TB = lax.dot_general(emb, mW0_ref[:, EMB:2 * EMB], dims11,
                         preferred_element_type=f32)
    TAB = jnp.concatenate([TA, TB], axis=0)
    h0 = (lax.dot_general(ohcat, TAB.astype(bf16), dims00,
                          preferred_element_type=f32)
          + lax.dot_general(eft_ref[...].astype(bf16),
                            mW0_ref[:, 2 * EMB:].astype(bf16), dims01,
                            preferred_element_type=f32)
          + b0_ref[...])
    h1 = jnp.maximum(h0, 0.0).astype(bf16)
    h2 = jnp.maximum(
        lax.dot_general(h1, mW1_ref[...].astype(bf16), dims11,
                        preferred_element_type=f32)
        + b1_ref[...], 0.0).astype(bf16)
    out_ref[...] = lax.dot_general(h2, mW2p_ref[...].astype(bf16), dims11,
                                   preferred_element_type=f32) + b2p_ref[...]


def _edge_mlp(cs3, cd3, eft, emb_s, mW0, b0r, mW1, b1r, mW2p, b2pr):
    nb = cs3.shape[0]
    ne = nb * _EB
    full = lambda shape: pl.BlockSpec(shape, lambda i, _s=shape: tuple(0 for _ in _s))
    return pl.pallas_call(
        _edge_body,
        grid=(nb,),
        in_specs=[
            pl.BlockSpec((1, 1, _EB), lambda i: (i, 0, 0)),
            pl.BlockSpec((1, 1, _EB), lambda i: (i, 0, 0)),
            pl.BlockSpec((DE, _EB), lambda i: (0, i)),
            full((10, EMB)),
            full((H, 2 * EMB + DE)),
            full((1, H)),
            full((H, H)),
            full((1, H)),
            full((128, H)),
            full((1, 128)),
        ],
        out_specs=pl.BlockSpec((_EB, 128), lambda i: (i, 0)),
        out_shape=jax.ShapeDtypeStruct((ne, 128), jnp.float32),
        compiler_params=pltpu.CompilerParams(
            dimension_semantics=("arbitrary",)),
    )(cs3, cd3, eft, emb_s, mW0, b0r, mW1, b1r, mW2p, b2pr)


_RB = 2304
_RNB = N // _RB


def _post_body(agg0_ref, agg1_ref, agg2_ref, pzt_ref, sht_ref,
               sct_ref, pW0_ref, pb0_ref, pW1_ref,
               pb1_ref, pW2_ref, pb2_ref, Wi_ref, Wf_ref, Wg_ref, Wo_ref,
               Ui_ref, Uf_ref, Ug_ref, Uo_ref, bi_ref, bf_ref, bg_ref,
               bo_ref, oW_ref, ob_ref, h_ref, c_ref, o_ref):
    f32 = jnp.float32
    dims11 = (((1,), (1,)), ((), ()))
    dims10 = (((1,), (0,)), ((), ()))

    bf16 = jnp.bfloat16

    def dott(w, x):
        return lax.dot_general(w.astype(bf16), x.astype(bf16), dims10,
                               preferred_element_type=f32)

    agg96 = agg0_ref[:, 0:H] + agg1_ref[:, 0:H] + agg2_ref[:, 0:H]
    g0 = (lax.dot_general(pW0_ref[:, 0:H].astype(bf16), agg96.astype(bf16),
                          dims11, preferred_element_type=f32)
          + dott(pW0_ref[:, H:H + EMB], pzt_ref[...]) + pb0_ref[...])
    h = jnp.maximum(g0, 0.0)
    h = jnp.maximum(dott(pW1_ref[...], h) + pb1_ref[...], 0.0)
    hp = dott(pW2_ref[...], h) + pb2_ref[...]
    sh = sht_ref[...]
    ii = jax.nn.sigmoid(dott(Wi_ref[...], hp) + dott(Ui_ref[...], sh) + bi_ref[...])
    ff = jax.nn.sigmoid(dott(Wf_ref[...], hp) + dott(Uf_ref[...], sh) + bf_ref[...])
    gg = jnp.tanh(dott(Wg_ref[...], hp) + dott(Ug_ref[...], sh) + bg_ref[...])
    oo = jax.nn.sigmoid(dott(Wo_ref[...], hp) + dott(Uo_ref[...], sh) + bo_ref[...])
    cn = ff * sct_ref[...] + ii * gg
    hn = oo * jnp.tanh(cn)
    h_ref[...] = hn
    c_ref[...] = cn
    o_ref[...] = dott(oW_ref[...], hn) + ob_ref[...]


def _post(aggs, puzzlet, sht, sct, pW0, pb0c, pW1, pb1c, pW2, pb2c, Ws, Us,
          bs, oW, obc):
    full = lambda shape: pl.BlockSpec(shape, lambda i, _s=shape: tuple(0 for _ in _s))
    return pl.pallas_call(
        _post_body,
        grid=(_RNB,),
        in_specs=[
            pl.BlockSpec((_RB, 128), lambda i: (i, 0)),
            pl.BlockSpec((_RB, 128), lambda i: (i, 0)),
            pl.BlockSpec((_RB, 128), lambda i: (i, 0)),
            pl.BlockSpec((EMB, _RB), lambda i: (0, i)),
            pl.BlockSpec((H, _RB), lambda i: (0, i)),
            pl.BlockSpec((H, _RB), lambda i: (0, i)),
            full((H, H + EMB)),
            full((H, 1)),
            full((H, H)),
            full((H, 1)),
            full((H, H)),
            full((H, 1)),
            *[full((H, H)) for _ in range(8)],
            *[full((H, 1)) for _ in range(4)],
            full((10, H)),
            full((10, 1)),
        ],
        out_specs=[
            pl.BlockSpec((H, _RB), lambda i: (0, i)),
            pl.BlockSpec((H, _RB), lambda i: (0, i)),
            pl.BlockSpec((10, _RB), lambda i: (0, i)),
        ],
        out_shape=[
            jax.ShapeDtypeStruct((H, N), jnp.float32),
            jax.ShapeDtypeStruct((H, N), jnp.float32),
            jax.ShapeDtypeStruct((10, N), jnp.float32),
        ],
        compiler_params=pltpu.CompilerParams(
            dimension_semantics=("arbitrary",)),
    )(*aggs, puzzlet, sht, sct, pW0, pb0c, pW1, pb1c, pW2, pb2c, *Ws, *Us,
      *bs, oW, obc)


def kernel(puzzle, nodes, edges, edge_features, state_h, state_c, first, emb,
           mW0, mb0, mW1, mb1, mW2, mb2, pW0, pb0, pW1, pb1, pW2, pb2, W_ih,
           W_hh, b_ih, b_hh, oW, ob):
    f32 = jnp.float32
    nodes = nodes.astype(jnp.int32)
    eflat = edges.astype(jnp.int32).reshape(2 * E)
    dst2d = edges[1].astype(jnp.int32).reshape(E // 128, 128)

    cls = _gather_cls(nodes, eflat)
    cs3 = cls[:E].reshape(_ENB, 1, _EB)
    cd3 = cls[E:].reshape(_ENB, 1, _EB)
    eft = edge_features.T

    emb_s = emb * jnp.asarray(first, f32)
    mW2p = jnp.concatenate([mW2, jnp.zeros((128 - H, H), f32)], axis=0)
    b2p = jnp.concatenate([mb2, jnp.zeros((128 - H,), f32)]).reshape(1, 128)

    aggs = []
    sup0 = 0
    for nsup in _CHUNKS:
        e0 = sup0 * _SUPER_ROWS * 128
        e1 = (sup0 + nsup) * _SUPER_ROWS * 128
        b0_, b1_ = e0 // _EB, e1 // _EB
        msgs_k = _edge_mlp(
            cs3[b0_:b1_], cd3[b0_:b1_],
            eft[:, e0:e1], emb_s, mW0, mb0.reshape(1, H), mW1,
            mb1.reshape(1, H), mW2p, b2p)
        aggs.append(_scatter_agg(dst2d, msgs_k, sup0, nsup))
        sup0 += nsup

    b = b_ih + b_hh
    Ws = [W_ih[i * H:(i + 1) * H] for i in range(4)]
    Us = [W_hh[i * H:(i + 1) * H] for i in range(4)]
    bs = [b[i * H:(i + 1) * H].reshape(H, 1) for i in range(4)]
    ht, ct, outt = _post(
        aggs, puzzle.T, state_h.T, state_c.T, pW0, pb0.reshape(H, 1), pW1,
        pb1.reshape(H, 1), pW2, pb2.reshape(H, 1), Ws, Us, bs, oW,
        ob.reshape(10, 1))
    return (ht.T, ct.T, outt.T.reshape(-1, 81, 10))


# speedup vs baseline: 1.0579x; 1.0473x over previous
import functools

import jax
import jax.numpy as jnp
from jax import lax
from jax.experimental import pallas as pl
from jax.experimental.pallas import tpu as pltpu
from jax.experimental.pallas import tpu_sc as plsc

N = 41472
E = 829440
H = 96
EMB = 16
DE = 16

_GW = 32
_GCHUNK = (2 * E) // _GW
_GSUB = 6480
_GNSUB = _GCHUNK // _GSUB


def _gather_cls(nodes, eflat):
    mesh = plsc.VectorSubcoreMesh(core_axis_name="c", subcore_axis_name="s")

    @functools.partial(
        pl.kernel,
        out_type=jax.ShapeDtypeStruct((2 * E,), jnp.int32),
        mesh=mesh,
        scratch_types=[
            pltpu.VMEM_SHARED((N,), jnp.int32),
            pltpu.VMEM((_GSUB,), jnp.int32),
            pltpu.VMEM((_GSUB,), jnp.int32),
            pltpu.VMEM((_GSUB,), jnp.int32),
            pltpu.VMEM((_GSUB,), jnp.int32),
            pltpu.SemaphoreType.DMA((2,)),
            pltpu.SemaphoreType.DMA,
        ],
    )
    def k(nodes_hbm, eflat_hbm, out_hbm, tbl, ibuf0, ibuf1, obuf0, obuf1,
          isem, gsem):
        ibufs = (ibuf0, ibuf1)
        obufs = (obuf0, obuf1)
        cid = lax.axis_index("c")
        sid = lax.axis_index("s")
        wid = sid * 2 + cid
        base = wid * _GCHUNK

        @pl.when(sid == 0)
        def _():
            pltpu.sync_copy(nodes_hbm, tbl)
        plsc.subcore_barrier()

        pltpu.async_copy(eflat_hbm.at[pl.ds(base, _GSUB)], ibufs[0],
                         isem.at[0])
        for sc in range(_GNSUB):
            b = sc % 2
            off = base + sc * _GSUB
            if sc + 1 < _GNSUB:
                pltpu.async_copy(
                    eflat_hbm.at[pl.ds(off + _GSUB, _GSUB)],
                    ibufs[1 - b], isem.at[1 - b])
            pltpu.make_async_copy(eflat_hbm.at[pl.ds(off, _GSUB)],
                                  ibufs[b], isem.at[b]).wait()
            pltpu.async_copy(tbl.at[ibufs[b]], obufs[b], gsem).wait()
            pltpu.sync_copy(obufs[b], out_hbm.at[pl.ds(off, _GSUB)])

    return k(nodes, eflat)


_SG = 24
_SCHUNK_ROWS = 9
_SCHUNK = _SCHUNK_ROWS * 128
_SUPER_ROWS = 16 * _SCHUNK_ROWS
_NSUPER = (E // 128) // _SUPER_ROWS
_CHUNKS = (15, 15, 15)
_SZROWS = 162
_NPT = N // 16


def _scatter_agg(dst2d, msgs2, super0, nsup):
    mesh = plsc.VectorSubcoreMesh(core_axis_name="c", subcore_axis_name="s")

    @functools.partial(
        pl.kernel,
        out_type=jax.ShapeDtypeStruct((N, 128), jnp.float32),
        mesh=mesh,
        scratch_types=[
            pltpu.VMEM_SHARED((N, _SG), jnp.float32),
            pltpu.VMEM((_SZROWS, _SG), jnp.float32),
            pltpu.VMEM((2, _SCHUNK, _SG), jnp.float32),
            pltpu.VMEM((2, _SCHUNK_ROWS, 128), jnp.int32),
            pltpu.SemaphoreType.DMA((2,)),
            pltpu.SemaphoreType.DMA((2,)),
            pltpu.SemaphoreType.DMA,
        ],
        compiler_params=pltpu.CompilerParams(use_tc_tiling_on_sc=False),
    )
    def k(dst2d_hbm, msgs2_hbm, out_hbm, acc, zbuf, dbuf, ibuf, dsem, isem,
          ssem):
        cid = lax.axis_index("c")
        sid = lax.axis_index("s")

        zeros16 = jnp.zeros((16,), jnp.float32)

        def zrow(i, carry):
            zbuf[i, pl.ds(0, 16)] = zeros16
            zbuf[i, pl.ds(8, 16)] = zeros16
            return carry

        lax.fori_loop(0, _SZROWS, zrow, 0)

        for p in range(2):
            grp = cid * 2 + p
            col0 = grp * _SG
            for t in range(_NPT // _SZROWS):
                pltpu.sync_copy(
                    zbuf, acc.at[pl.ds(sid * _NPT + t * _SZROWS, _SZROWS)])
            plsc.subcore_barrier()

            def start_in(t, b):
                lrow0 = t * _SUPER_ROWS + sid * _SCHUNK_ROWS
                grow0 = super0 * _SUPER_ROWS + lrow0
                pltpu.async_copy(
                    msgs2_hbm.at[pl.ds(lrow0 * 128, _SCHUNK),
                                 pl.ds(col0, _SG)],
                    dbuf.at[b], dsem.at[b])
                pltpu.async_copy(dst2d_hbm.at[pl.ds(grow0, _SCHUNK_ROWS)],
                                 ibuf.at[b], isem.at[b])

            def wait_in(t, b):
                lrow0 = t * _SUPER_ROWS + sid * _SCHUNK_ROWS
                grow0 = super0 * _SUPER_ROWS + lrow0
                pltpu.make_async_copy(
                    msgs2_hbm.at[pl.ds(lrow0 * 128, _SCHUNK),
                                 pl.ds(col0, _SG)],
                    dbuf.at[b], dsem.at[b]).wait()
                pltpu.make_async_copy(
                    dst2d_hbm.at[pl.ds(grow0, _SCHUNK_ROWS)],
                    ibuf.at[b], isem.at[b]).wait()

            start_in(0, 0)

            def chunk2(t2, carry):
                for b in range(2):
                    t = t2 * 2 + b

                    @pl.when(t < nsup)
                    def _():
                        @pl.when(t + 1 < nsup)
                        def _():
                            start_in(t + 1, 1 - b)
                        wait_in(t, b)
                        descs = []
                        for j in range(_SCHUNK_ROWS):
                            descs.append(pltpu.async_copy(
                                dbuf.at[b, pl.ds(j * 128, 128)],
                                acc.at[ibuf.at[b, j]], ssem, add=True))
                        for d in descs:
                            d.wait()
                return carry

            lax.fori_loop(0, (nsup + 1) // 2, chunk2, 0)
            plsc.subcore_barrier()

            pltpu.sync_copy(acc.at[pl.ds(sid * _NPT, _NPT)],
                            out_hbm.at[pl.ds(sid * _NPT, _NPT),
                                       pl.ds(col0, _SG)])

    return k(dst2d, msgs2)


_EB = 10240
_ENB = E // _EB


def _edge_body(cs_ref, cd_ref, eft_ref, emb_ref, mW0_ref, b0_ref, mW1_ref,
               b1_ref, mW2p_ref, b2p_ref, out_ref):
    f32 = jnp.float32
    bf16 = jnp.bfloat16
    dims11 = (((1,), (1,)), ((), ()))
    dims00 = (((0,), (0,)), ((), ()))
    dims01 = (((0,), (1,)), ((), ()))
    cs = cs_ref[0]
    cd = cd_ref[0]
    iota10c = lax.broadcasted_iota(jnp.int32, (10, 1), 0)
    oh_st = (cs == iota10c).astype(bf16)
    oh_dt = (cd == iota10c).astype(bf16)
    ohcat = jnp.concatenate([oh_st, oh_dt], axis=0)
    emb = emb_ref[...]
    TA = lax.dot_general(emb, mW0_ref[:, 0:EMB], dims11,
                         preferred_element_type=f32)
    TB = lax.dot_general(emb, mW0_ref[:, EMB:2 * EMB], dims11,
                         preferred_element_type=f32)
    TAB = jnp.concatenate([TA, TB], axis=0)
    h0 = (lax.dot_general(ohcat, TAB.astype(bf16), dims00,
                          preferred_element_type=f32)
          + lax.dot_general(eft_ref[...].astype(bf16),
                            mW0_ref[:, 2 * EMB:].astype(bf16), dims01,
                            preferred_element_type=f32)
          + b0_ref[...])
    h1 = jnp.maximum(h0, 0.0).astype(bf16)
    h2 = jnp.maximum(
        lax.dot_general(h1, mW1_ref[...].astype(bf16), dims11,
                        preferred_element_type=f32)
        + b1_ref[...], 0.0).astype(bf16)
    out_ref[...] = lax.dot_general(h2, mW2p_ref[...].astype(bf16), dims11,
                                   preferred_element_type=f32) + b2p_ref[...]


def _edge_mlp(cs3, cd3, eft, emb_s, mW0, b0r, mW1, b1r, mW2p, b2pr):
    nb = cs3.shape[0]
    ne = nb * _EB
    full = lambda shape: pl.BlockSpec(shape, lambda i, _s=shape: tuple(0 for _ in _s))
    return pl.pallas_call(
        _edge_body,
        grid=(nb,),
        in_specs=[
            pl.BlockSpec((1, 1, _EB), lambda i: (i, 0, 0)),
            pl.BlockSpec((1, 1, _EB), lambda i: (i, 0, 0)),
            pl.BlockSpec((DE, _EB), lambda i: (0, i)),
            full((10, EMB)),
            full((H, 2 * EMB + DE)),
            full((1, H)),
            full((H, H)),
            full((1, H)),
            full((128, H)),
            full((1, 128)),
        ],
        out_specs=pl.BlockSpec((_EB, 128), lambda i: (i, 0)),
        out_shape=jax.ShapeDtypeStruct((ne, 128), jnp.float32),
        compiler_params=pltpu.CompilerParams(
            dimension_semantics=("arbitrary",)),
    )(cs3, cd3, eft, emb_s, mW0, b0r, mW1, b1r, mW2p, b2pr)


_RB = 2304
_RNB = N // _RB


def _post_body(agg0_ref, agg1_ref, agg2_ref, pzt_ref, sht_ref,
               sct_ref, pW0_ref, pb0_ref, pW1_ref,
               pb1_ref, pW2_ref, pb2_ref, Wi_ref, Wf_ref, Wg_ref, Wo_ref,
               Ui_ref, Uf_ref, Ug_ref, Uo_ref, bi_ref, bf_ref, bg_ref,
               bo_ref, oW_ref, ob_ref, h_ref, c_ref, o_ref):
    f32 = jnp.float32
    dims11 = (((1,), (1,)), ((), ()))
    dims10 = (((1,), (0,)), ((), ()))

    bf16 = jnp.bfloat16

    def dott(w, x):
        return lax.dot_general(w.astype(bf16), x.astype(bf16), dims10,
                               preferred_element_type=f32)

    agg96 = agg0_ref[:, 0:H] + agg1_ref[:, 0:H] + agg2_ref[:, 0:H]
    g0 = (lax.dot_general(pW0_ref[:, 0:H].astype(bf16), agg96.astype(bf16),
                          dims11, preferred_element_type=f32)
          + dott(pW0_ref[:, H:H + EMB], pzt_ref[...]) + pb0_ref[...])
    h = jnp.maximum(g0, 0.0)
    h = jnp.maximum(dott(pW1_ref[...], h) + pb1_ref[...], 0.0)
    hp = dott(pW2_ref[...], h) + pb2_ref[...]
    sh = sht_ref[...]
    ii = jax.nn.sigmoid(dott(Wi_ref[...], hp) + dott(Ui_ref[...], sh) + bi_ref[...])
    ff = jax.nn.sigmoid(dott(Wf_ref[...], hp) + dott(Uf_ref[...], sh) + bf_ref[...])
    gg = jnp.tanh(dott(Wg_ref[...], hp) + dott(Ug_ref[...], sh) + bg_ref[...])
    oo = jax.nn.sigmoid(dott(Wo_ref[...], hp) + dott(Uo_ref[...], sh) + bo_ref[...])
    cn = ff * sct_ref[...] + ii * gg
    hn = oo * jnp.tanh(cn)
    h_ref[...] = hn
    c_ref[...] = cn
    o_ref[...] = dott(oW_ref[...], hn) + ob_ref[...]


def _post(aggs, puzzlet, sht, sct, pW0, pb0c, pW1, pb1c, pW2, pb2c, Ws, Us,
          bs, oW, obc):
    full = lambda shape: pl.BlockSpec(shape, lambda i, _s=shape: tuple(0 for _ in _s))
    return pl.pallas_call(
        _post_body,
        grid=(_RNB,),
        in_specs=[
            pl.BlockSpec((_RB, 128), lambda i: (i, 0)),
            pl.BlockSpec((_RB, 128), lambda i: (i, 0)),
            pl.BlockSpec((_RB, 128), lambda i: (i, 0)),
            pl.BlockSpec((EMB, _RB), lambda i: (0, i)),
            pl.BlockSpec((H, _RB), lambda i: (0, i)),
            pl.BlockSpec((H, _RB), lambda i: (0, i)),
            full((H, H + EMB)),
            full((H, 1)),
            full((H, H)),
            full((H, 1)),
            full((H, H)),
            full((H, 1)),
            *[full((H, H)) for _ in range(8)],
            *[full((H, 1)) for _ in range(4)],
            full((10, H)),
            full((10, 1)),
        ],
        out_specs=[
            pl.BlockSpec((H, _RB), lambda i: (0, i)),
            pl.BlockSpec((H, _RB), lambda i: (0, i)),
            pl.BlockSpec((10, _RB), lambda i: (0, i)),
        ],
        out_shape=[
            jax.ShapeDtypeStruct((H, N), jnp.float32),
            jax.ShapeDtypeStruct((H, N), jnp.float32),
            jax.ShapeDtypeStruct((10, N), jnp.float32),
        ],
        compiler_params=pltpu.CompilerParams(
            dimension_semantics=("arbitrary",)),
    )(*aggs, puzzlet, sht, sct, pW0, pb0c, pW1, pb1c, pW2, pb2c, *Ws, *Us,
      *bs, oW, obc)


def kernel(puzzle, nodes, edges, edge_features, state_h, state_c, first, emb,
           mW0, mb0, mW1, mb1, mW2, mb2, pW0, pb0, pW1, pb1, pW2, pb2, W_ih,
           W_hh, b_ih, b_hh, oW, ob):
    f32 = jnp.float32
    nodes = nodes.astype(jnp.int32)
    eflat = edges.astype(jnp.int32).reshape(2 * E)
    dst2d = edges[1].astype(jnp.int32).reshape(E // 128, 128)

    cls = _gather_cls(nodes, eflat)
    cs3 = cls[:E].reshape(_ENB, 1, _EB)
    cd3 = cls[E:].reshape(_ENB, 1, _EB)
    eft = edge_features.T

    emb_s = emb * jnp.asarray(first, f32)
    mW2p = jnp.concatenate([mW2, jnp.zeros((128 - H, H), f32)], axis=0)
    b2p = jnp.concatenate([mb2, jnp.zeros((128 - H,), f32)]).reshape(1, 128)

    aggs = []
    sup0 = 0
    for nsup in _CHUNKS:
        e0 = sup0 * _SUPER_ROWS * 128
        e1 = (sup0 + nsup) * _SUPER_ROWS * 128
        b0_, b1_ = e0 // _EB, e1 // _EB
        msgs_k = _edge_mlp(
            cs3[b0_:b1_], cd3[b0_:b1_],
            eft[:, e0:e1], emb_s, mW0, mb0.reshape(1, H), mW1,
            mb1.reshape(1, H), mW2p, b2p)
        aggs.append(_scatter_agg(dst2d, msgs_k, sup0, nsup))
        sup0 += nsup

    b = b_ih + b_hh
    Ws = [W_ih[i * H:(i + 1) * H] for i in range(4)]
    Us = [W_hh[i * H:(i + 1) * H] for i in range(4)]
    bs = [b[i * H:(i + 1) * H].reshape(H, 1) for i in range(4)]
    ht, ct, outt = _post(
        aggs, puzzle.T, state_h.T, state_c.T, pW0, pb0.reshape(H, 1), pW1,
        pb1.reshape(H, 1), pW2, pb2.reshape(H, 1), Ws, Us, bs, oW,
        ob.reshape(10, 1))
    return (ht.T, ct.T, outt.T.reshape(-1, 81, 10))
